# baseline (speedup 1.0000x reference)
import functools

import jax
import jax.numpy as jnp
from jax import lax
from jax.experimental import pallas as pl
from jax.experimental.pallas import tpu as pltpu
from jax.experimental.pallas import tpu_sc as plsc

LOG2E = 1.4426950408889634

_R, _C = 512, 2048


def _fused_body(cn_ref, cnl2_ref, z_ref, cb_ref, cbb_ref,
                idx_ref, summ_ref, ent_ref,
                zs_ref, zbb_ref, zn_ref, m_ref, s_ref, fidx_ref, mr_ref,
                pacc_ref,
                *, n_i, n_j, c_blk, k_total, n_rows):
    i = pl.program_id(0)
    s = pl.program_id(1)
    j = pl.program_id(2)

    @pl.when((s == 0) & (j == 0))
    def _():
        zt = z_ref[...]
        zs_ref[...] = zt * (-2.0)
        zbb_ref[...] = (zt * (-2.0 * LOG2E)).astype(jnp.bfloat16)
        zn_ref[...] = jnp.sum(zt * zt, axis=1, keepdims=True)

    @pl.when(s == 0)
    def _():
        t = lax.dot_general(zs_ref[...], cb_ref[...],
                            dimension_numbers=(((1,), (1,)), ((), ())),
                            preferred_element_type=jnp.float32)
        d = (zn_ref[...] + t) + cn_ref[...]
        lmin = jnp.min(d, axis=1, keepdims=True)
        g = (lmin - d) * LOG2E
        jg = (lax.broadcasted_iota(jnp.int32, d.shape, 1)
              .astype(jnp.float32) + jnp.float32(j * c_blk))
        larg = jnp.min(jnp.where(g == 0.0, jg, jnp.float32(2 * k_total)),
                       axis=1, keepdims=True)
        ls = jnp.sum(jnp.exp2(g), axis=1, keepdims=True)

        @pl.when(j == 0)
        def _():
            m_ref[...] = lmin
            fidx_ref[...] = larg
            s_ref[...] = ls

        @pl.when(j > 0)
        def _():
            mo = m_ref[...]
            so = s_ref[...]
            mn = jnp.minimum(mo, lmin)
            s_ref[...] = (so * jnp.exp2((mn - mo) * LOG2E)
                          + ls * jnp.exp2((mn - lmin) * LOG2E))
            m_ref[...] = mn
            fidx_ref[...] = jnp.where(lmin < mo, larg, fidx_ref[...])

        @pl.when(j == n_j - 1)
        def _():
            part = jnp.sum(m_ref[...])
            idx_ref[...] = fidx_ref[...].astype(jnp.int32)
            mr_ref[...] = ((m_ref[...] - zn_ref[...])
                           - jnp.log(s_ref[...])) * LOG2E

            @pl.when(i == 0)
            def _():
                summ_ref[0, 0] = part

            @pl.when(i > 0)
            def _():
                summ_ref[0, 0] = summ_ref[0, 0] + part

    @pl.when(s == 1)
    def _():
        t = lax.dot_general(zbb_ref[...], cbb_ref[...],
                            dimension_numbers=(((1,), (1,)), ((), ())),
                            preferred_element_type=jnp.float32)
        w = jnp.exp2(mr_ref[...] - (t + cnl2_ref[...]))
        cs = jnp.sum(w, axis=0, keepdims=True)

        @pl.when(i == 0)
        def _():
            pacc_ref[j, :, :] = cs

        @pl.when(i > 0)
        def _():
            pacc_ref[j, :, :] = pacc_ref[j, :, :] + cs

        @pl.when((i == n_i - 1) & (j == n_j - 1))
        def _():
            p = pacc_ref[...] * (1.0 / n_rows)
            ent_ref[0, 0] = -jnp.sum(p * jnp.log(p + 1e-10))


def _run_fused(z2, codebook, cn, cnl2, cbb, r_blk, c_blk, interpret=False):
    n, d_model = z2.shape
    kcb = codebook.shape[0]
    n_i, n_j = n // r_blk, kcb // c_blk
    return pl.pallas_call(
        functools.partial(_fused_body, n_i=n_i, n_j=n_j, c_blk=c_blk,
                          k_total=kcb, n_rows=n),
        grid=(n_i, 2, n_j),
        in_specs=[
            pl.BlockSpec((1, c_blk), lambda i, s, j: (0, j)),
            pl.BlockSpec((1, c_blk), lambda i, s, j: (0, j)),
            pl.BlockSpec((r_blk, d_model), lambda i, s, j: (i, 0)),
            pl.BlockSpec((c_blk, d_model),
                         lambda i, s, j: (jnp.where(s == 0, j, 0), 0)),
            pl.BlockSpec((c_blk, d_model),
                         lambda i, s, j: (jnp.where(s == 1, j, 0), 0)),
        ],
        out_specs=[
            pl.BlockSpec((r_blk, 1), lambda i, s, j: (i, 0)),
            pl.BlockSpec((1, 1), lambda i, s, j: (0, 0),
                         memory_space=pltpu.SMEM),
            pl.BlockSpec((1, 1), lambda i, s, j: (0, 0),
                         memory_space=pltpu.SMEM),
        ],
        out_shape=[
            jax.ShapeDtypeStruct((n, 1), jnp.int32),
            jax.ShapeDtypeStruct((1, 1), jnp.float32),
            jax.ShapeDtypeStruct((1, 1), jnp.float32),
        ],
        scratch_shapes=[
            pltpu.VMEM((r_blk, d_model), jnp.float32),
            pltpu.VMEM((r_blk, d_model), jnp.bfloat16),
            pltpu.VMEM((r_blk, 1), jnp.float32),
            pltpu.VMEM((r_blk, 1), jnp.float32),
            pltpu.VMEM((r_blk, 1), jnp.float32),
            pltpu.VMEM((r_blk, 1), jnp.float32),
            pltpu.VMEM((r_blk, 1), jnp.float32),
            pltpu.VMEM((n_j, 1, c_blk), jnp.float32),
        ],
        interpret=interpret,
    )(cn, cnl2, z2, codebook, cbb)


def _make_sc_gather(n_rows, d_model, n_workers, chunk):
    b_per_w = n_rows // n_workers
    n_chunks = b_per_w // chunk
    mesh = plsc.VectorSubcoreMesh(core_axis_name="c", subcore_axis_name="s")

    @functools.partial(
        pl.kernel, mesh=mesh,
        out_type=jax.ShapeDtypeStruct((n_rows, d_model), jnp.float32),
        scratch_types=[
            pltpu.VMEM((b_per_w,), jnp.int32),
            pltpu.VMEM((chunk, d_model), jnp.float32),
            pltpu.VMEM((chunk, d_model), jnp.float32),
            pltpu.SemaphoreType.DMA,
            pltpu.SemaphoreType.DMA,
        ],
    )
    def gather_k(cb_hbm, idx_hbm, out_hbm, idx_v, rows_a, rows_b, sem_a, sem_b):
        wid = lax.axis_index("s") * 2 + lax.axis_index("c")
        base = wid * b_per_w
        pltpu.sync_copy(idx_hbm.at[pl.ds(base, b_per_w)], idx_v)
        bufs = ((rows_a, sem_a), (rows_b, sem_b))
        cps = []
        for c in range(n_chunks):
            buf, sem = bufs[c % 2]
            cps.append(pltpu.async_copy(
                cb_hbm.at[idx_v.at[pl.ds(c * chunk, chunk)]], buf, sem))
            if c >= 1:
                cps[c - 1].wait()
                pbuf, _ = bufs[(c - 1) % 2]
                pltpu.sync_copy(
                    pbuf, out_hbm.at[pl.ds(base + (c - 1) * chunk, chunk)])
        cps[n_chunks - 1].wait()
        lbuf, _ = bufs[(n_chunks - 1) % 2]
        pltpu.sync_copy(
            lbuf, out_hbm.at[pl.ds(base + (n_chunks - 1) * chunk, chunk)])

    return gather_k


def kernel(z, codebook):
    b, k_seq, d_model = z.shape
    n = b * k_seq
    kcb = codebook.shape[0]
    z2 = z.reshape(n, d_model)

    cn = jnp.sum(codebook ** 2, axis=1)[None, :]
    cnl2 = cn * LOG2E
    cbb = codebook.astype(jnp.bfloat16)

    idx, summ, ent = _run_fused(z2, codebook, cn, cnl2, cbb, _R, _C)

    gather_k = _make_sc_gather(n, d_model, 32, 128)
    z_q = gather_k(codebook, idx.reshape(n))

    sum_min = summ[0, 0]
    entropy = ent[0, 0]
    max_ent = jnp.log(jnp.float32(kcb))
    total_loss = (1.25 * sum_min / jnp.float32(n * d_model)
                  + 0.1 * (max_ent - entropy) / max_ent)
    return (z_q.reshape(b, k_seq, d_model), total_loss,
            idx.reshape(b, k_seq))


# speedup vs baseline: 1.0671x; 1.0671x over previous
import functools

import jax
import jax.numpy as jnp
from jax import lax
from jax.experimental import pallas as pl
from jax.experimental.pallas import tpu as pltpu
from jax.experimental.pallas import tpu_sc as plsc

LOG2E = 1.4426950408889634

_R1, _C1 = 512, 2048
_R2, _C2 = 512, 2048


def _p1_body(cn_ref, z_ref, cb_ref, idx_ref, mr_ref, summ_ref, zbb_ref,
             zs_ref, zn_ref, m_ref, s_ref, fidx_ref,
             *, n_i, n_j, c_blk, k_total):
    i = pl.program_id(0)
    j = pl.program_id(1)

    @pl.when(j == 0)
    def _():
        zt = z_ref[...]
        zs_ref[...] = zt * (-2.0)
        zbb_ref[...] = (zt * (-2.0 * LOG2E)).astype(jnp.bfloat16)
        zn_ref[...] = jnp.sum(zt * zt, axis=1, keepdims=True)

    t = lax.dot_general(zs_ref[...], cb_ref[...],
                        dimension_numbers=(((1,), (1,)), ((), ())),
                        preferred_element_type=jnp.float32)
    d = (zn_ref[...] + t) + cn_ref[...]
    lmin = jnp.min(d, axis=1, keepdims=True)
    g = (lmin - d) * LOG2E
    jg = (lax.broadcasted_iota(jnp.int32, d.shape, 1).astype(jnp.float32)
          + jnp.float32(j * c_blk))
    larg = jnp.min(jnp.where(g == 0.0, jg, jnp.float32(2 * k_total)),
                   axis=1, keepdims=True)
    ls = jnp.sum(jnp.exp2(g), axis=1, keepdims=True)

    @pl.when(j == 0)
    def _():
        m_ref[...] = lmin
        fidx_ref[...] = larg
        s_ref[...] = ls

    @pl.when(j > 0)
    def _():
        mo = m_ref[...]
        so = s_ref[...]
        mn = jnp.minimum(mo, lmin)
        s_ref[...] = (so * jnp.exp2((mn - mo) * LOG2E)
                      + ls * jnp.exp2((mn - lmin) * LOG2E))
        m_ref[...] = mn
        fidx_ref[...] = jnp.where(lmin < mo, larg, fidx_ref[...])

    @pl.when(j == n_j - 1)
    def _():
        part = jnp.sum(m_ref[...])
        idx_ref[...] = fidx_ref[...].astype(jnp.int32)
        mr_ref[...] = ((m_ref[...] - zn_ref[...])
                       - jnp.log(s_ref[...])) * LOG2E

        @pl.when(i == 0)
        def _():
            summ_ref[0, 0] = part

        @pl.when(i > 0)
        def _():
            summ_ref[0, 0] = summ_ref[0, 0] + part


def _p2_body(cnl2_ref, mr_ref, z_ref, cb_ref, ent_ref, acc_ref,
             *, n_i, n_j, n_rows):
    j = pl.program_id(0)
    i = pl.program_id(1)
    t = lax.dot_general(z_ref[...], cb_ref[...],
                        dimension_numbers=(((1,), (1,)), ((), ())),
                        preferred_element_type=jnp.float32)
    w = jnp.exp2(mr_ref[...] - (t + cnl2_ref[...]))
    cs = jnp.sum(w, axis=0, keepdims=True)

    @pl.when(i == 0)
    def _():
        acc_ref[...] = cs

    @pl.when(i > 0)
    def _():
        acc_ref[...] = acc_ref[...] + cs

    @pl.when(i == n_i - 1)
    def _():
        p = acc_ref[...] * (1.0 / n_rows)
        part = -jnp.sum(p * jnp.log(p + 1e-10))

        @pl.when(j == 0)
        def _():
            ent_ref[0, 0] = part

        @pl.when(j > 0)
        def _():
            ent_ref[0, 0] = ent_ref[0, 0] + part


def _run_p1(z2, codebook, cn, r_blk, c_blk, interpret=False):
    n, d_model = z2.shape
    kcb = codebook.shape[0]
    n_i, n_j = n // r_blk, kcb // c_blk
    return pl.pallas_call(
        functools.partial(_p1_body, n_i=n_i, n_j=n_j, c_blk=c_blk,
                          k_total=kcb),
        grid=(n_i, n_j),
        in_specs=[
            pl.BlockSpec((1, c_blk), lambda i, j: (0, j)),
            pl.BlockSpec((r_blk, d_model), lambda i, j: (i, 0)),
            pl.BlockSpec((c_blk, d_model), lambda i, j: (j, 0)),
        ],
        out_specs=[
            pl.BlockSpec((r_blk, 1), lambda i, j: (i, 0)),
            pl.BlockSpec((r_blk, 1), lambda i, j: (i, 0)),
            pl.BlockSpec((1, 1), lambda i, j: (0, 0),
                         memory_space=pltpu.SMEM),
            pl.BlockSpec((r_blk, d_model), lambda i, j: (i, 0)),
        ],
        out_shape=[
            jax.ShapeDtypeStruct((n, 1), jnp.int32),
            jax.ShapeDtypeStruct((n, 1), jnp.float32),
            jax.ShapeDtypeStruct((1, 1), jnp.float32),
            jax.ShapeDtypeStruct((n, d_model), jnp.bfloat16),
        ],
        scratch_shapes=[
            pltpu.VMEM((r_blk, d_model), jnp.float32),
            pltpu.VMEM((r_blk, 1), jnp.float32),
            pltpu.VMEM((r_blk, 1), jnp.float32),
            pltpu.VMEM((r_blk, 1), jnp.float32),
            pltpu.VMEM((r_blk, 1), jnp.float32),
        ],
        interpret=interpret,
    )(cn, z2, codebook)


def _run_p2(zbb, cbb, cnl2, mr, r_blk, c_blk, interpret=False):
    n, d_model = zbb.shape
    kcb = cbb.shape[0]
    n_i, n_j = n // r_blk, kcb // c_blk
    return pl.pallas_call(
        functools.partial(_p2_body, n_i=n_i, n_j=n_j, n_rows=n),
        grid=(n_j, n_i),
        in_specs=[
            pl.BlockSpec((1, c_blk), lambda j, i: (0, j)),
            pl.BlockSpec((r_blk, 1), lambda j, i: (i, 0)),
            pl.BlockSpec((r_blk, d_model), lambda j, i: (i, 0)),
            pl.BlockSpec((c_blk, d_model), lambda j, i: (j, 0)),
        ],
        out_specs=pl.BlockSpec((1, 1), lambda j, i: (0, 0),
                               memory_space=pltpu.SMEM),
        out_shape=jax.ShapeDtypeStruct((1, 1), jnp.float32),
        scratch_shapes=[pltpu.VMEM((1, c_blk), jnp.float32)],
        interpret=interpret,
    )(cnl2, mr, zbb, cbb)


def _make_sc_gather(n_rows, d_model, n_workers, chunk):
    b_per_w = n_rows // n_workers
    n_chunks = b_per_w // chunk
    mesh = plsc.VectorSubcoreMesh(core_axis_name="c", subcore_axis_name="s")

    @functools.partial(
        pl.kernel, mesh=mesh,
        out_type=jax.ShapeDtypeStruct((n_rows, d_model), jnp.float32),
        scratch_types=[
            pltpu.VMEM((b_per_w,), jnp.int32),
            pltpu.VMEM((chunk, d_model), jnp.float32),
            pltpu.VMEM((chunk, d_model), jnp.float32),
            pltpu.SemaphoreType.DMA,
            pltpu.SemaphoreType.DMA,
        ],
    )
    def gather_k(cb_hbm, idx_hbm, out_hbm, idx_v, rows_a, rows_b, sem_a, sem_b):
        wid = lax.axis_index("s") * 2 + lax.axis_index("c")
        base = wid * b_per_w
        pltpu.sync_copy(idx_hbm.at[pl.ds(base, b_per_w)], idx_v)
        bufs = ((rows_a, sem_a), (rows_b, sem_b))
        cps = []
        for c in range(n_chunks):
            buf, sem = bufs[c % 2]
            cps.append(pltpu.async_copy(
                cb_hbm.at[idx_v.at[pl.ds(c * chunk, chunk)]], buf, sem))
            if c >= 1:
                cps[c - 1].wait()
                pbuf, _ = bufs[(c - 1) % 2]
                pltpu.sync_copy(
                    pbuf, out_hbm.at[pl.ds(base + (c - 1) * chunk, chunk)])
        cps[n_chunks - 1].wait()
        lbuf, _ = bufs[(n_chunks - 1) % 2]
        pltpu.sync_copy(
            lbuf, out_hbm.at[pl.ds(base + (n_chunks - 1) * chunk, chunk)])

    return gather_k


def kernel(z, codebook):
    b, k_seq, d_model = z.shape
    n = b * k_seq
    kcb = codebook.shape[0]
    z2 = z.reshape(n, d_model)

    cn = jnp.sum(codebook ** 2, axis=1)[None, :]
    cnl2 = cn * LOG2E
    cbb = codebook.astype(jnp.bfloat16)

    idx, mr, summ, zbb = _run_p1(z2, codebook, cn, _R1, _C1)

    gather_k = _make_sc_gather(n, d_model, 32, 128)
    z_q = gather_k(codebook, idx.reshape(n))

    ent = _run_p2(zbb, cbb, cnl2, mr, _R2, _C2)

    sum_min = summ[0, 0]
    entropy = ent[0, 0]
    max_ent = jnp.log(jnp.float32(kcb))
    total_loss = (1.25 * sum_min / jnp.float32(n * d_model)
                  + 0.1 * (max_ent - entropy) / max_ent)
    return (z_q.reshape(b, k_seq, d_model), total_loss,
            idx.reshape(b, k_seq))


# speedup vs baseline: 1.2177x; 1.1411x over previous
import functools

import jax
import jax.numpy as jnp
from jax import lax
from jax.experimental import pallas as pl
from jax.experimental.pallas import tpu as pltpu
from jax.experimental.pallas import tpu_sc as plsc

LOG2E = 1.4426950408889634

_R1, _C1 = 512, 4096
_R2, _C2 = 512, 4096


def _p1_body(cn_ref, z_ref, cb_ref, idx_ref, mr_ref, summ_ref, zbb_ref,
             zs_ref, zn_ref, m_ref, s_ref, fidx_ref,
             *, n_i, n_j, c_blk, k_total):
    i = pl.program_id(0)
    j = pl.program_id(1)

    @pl.when(j == 0)
    def _():
        zt = z_ref[...]
        zs_ref[...] = zt * (-2.0)
        zbb_ref[...] = (zt * (-2.0 * LOG2E)).astype(jnp.bfloat16)
        zn_ref[...] = jnp.sum(zt * zt, axis=1, keepdims=True)

    t = lax.dot_general(zs_ref[...], cb_ref[...],
                        dimension_numbers=(((1,), (1,)), ((), ())),
                        preferred_element_type=jnp.float32)
    d = (zn_ref[...] + t) + cn_ref[...]
    lmin = jnp.min(d, axis=1, keepdims=True)
    g = (lmin - d) * LOG2E
    jg = (lax.broadcasted_iota(jnp.int32, d.shape, 1).astype(jnp.float32)
          + jnp.float32(j * c_blk))
    larg = jnp.min(jnp.where(g == 0.0, jg, jnp.float32(2 * k_total)),
                   axis=1, keepdims=True)
    ls = jnp.sum(jnp.exp2(g), axis=1, keepdims=True)

    @pl.when(j == 0)
    def _():
        m_ref[...] = lmin
        fidx_ref[...] = larg
        s_ref[...] = ls

    @pl.when(j > 0)
    def _():
        mo = m_ref[...]
        so = s_ref[...]
        mn = jnp.minimum(mo, lmin)
        s_ref[...] = (so * jnp.exp2((mn - mo) * LOG2E)
                      + ls * jnp.exp2((mn - lmin) * LOG2E))
        m_ref[...] = mn
        fidx_ref[...] = jnp.where(lmin < mo, larg, fidx_ref[...])

    @pl.when(j == n_j - 1)
    def _():
        part = jnp.sum(m_ref[...])
        idx_ref[...] = fidx_ref[...].astype(jnp.int32)
        mr_ref[...] = ((m_ref[...] - zn_ref[...])
                       - jnp.log(s_ref[...])) * LOG2E

        @pl.when(i == 0)
        def _():
            summ_ref[0, 0] = part

        @pl.when(i > 0)
        def _():
            summ_ref[0, 0] = summ_ref[0, 0] + part


def _p2_body(cnl2_ref, mr_ref, z_ref, cb_ref, ent_ref, acc_ref,
             *, n_i, n_j, n_rows):
    j = pl.program_id(0)
    i = pl.program_id(1)
    t = lax.dot_general(z_ref[...], cb_ref[...],
                        dimension_numbers=(((1,), (1,)), ((), ())),
                        preferred_element_type=jnp.float32)
    w = jnp.exp2(mr_ref[...] - (t + cnl2_ref[...]))
    cs = jnp.sum(w, axis=0, keepdims=True)

    @pl.when(i == 0)
    def _():
        acc_ref[...] = cs

    @pl.when(i > 0)
    def _():
        acc_ref[...] = acc_ref[...] + cs

    @pl.when(i == n_i - 1)
    def _():
        p = acc_ref[...] * (1.0 / n_rows)
        part = -jnp.sum(p * jnp.log(p + 1e-10))

        @pl.when(j == 0)
        def _():
            ent_ref[0, 0] = part

        @pl.when(j > 0)
        def _():
            ent_ref[0, 0] = ent_ref[0, 0] + part


def _run_p1(z2, codebook, cn, r_blk, c_blk, interpret=False):
    n, d_model = z2.shape
    kcb = codebook.shape[0]
    n_i, n_j = n // r_blk, kcb // c_blk
    return pl.pallas_call(
        functools.partial(_p1_body, n_i=n_i, n_j=n_j, c_blk=c_blk,
                          k_total=kcb),
        grid=(n_i, n_j),
        in_specs=[
            pl.BlockSpec((1, c_blk), lambda i, j: (0, j)),
            pl.BlockSpec((r_blk, d_model), lambda i, j: (i, 0)),
            pl.BlockSpec((c_blk, d_model), lambda i, j: (j, 0)),
        ],
        out_specs=[
            pl.BlockSpec((r_blk, 1), lambda i, j: (i, 0)),
            pl.BlockSpec((r_blk, 1), lambda i, j: (i, 0)),
            pl.BlockSpec((1, 1), lambda i, j: (0, 0),
                         memory_space=pltpu.SMEM),
            pl.BlockSpec((r_blk, d_model), lambda i, j: (i, 0)),
        ],
        out_shape=[
            jax.ShapeDtypeStruct((n, 1), jnp.int32),
            jax.ShapeDtypeStruct((n, 1), jnp.float32),
            jax.ShapeDtypeStruct((1, 1), jnp.float32),
            jax.ShapeDtypeStruct((n, d_model), jnp.bfloat16),
        ],
        scratch_shapes=[
            pltpu.VMEM((r_blk, d_model), jnp.float32),
            pltpu.VMEM((r_blk, 1), jnp.float32),
            pltpu.VMEM((r_blk, 1), jnp.float32),
            pltpu.VMEM((r_blk, 1), jnp.float32),
            pltpu.VMEM((r_blk, 1), jnp.float32),
        ],
        interpret=interpret,
    )(cn, z2, codebook)


def _run_p2(zbb, cbb, cnl2, mr, r_blk, c_blk, interpret=False):
    n, d_model = zbb.shape
    kcb = cbb.shape[0]
    n_i, n_j = n // r_blk, kcb // c_blk
    return pl.pallas_call(
        functools.partial(_p2_body, n_i=n_i, n_j=n_j, n_rows=n),
        grid=(n_j, n_i),
        in_specs=[
            pl.BlockSpec((1, c_blk), lambda j, i: (0, j)),
            pl.BlockSpec((r_blk, 1), lambda j, i: (i, 0)),
            pl.BlockSpec((r_blk, d_model), lambda j, i: (i, 0)),
            pl.BlockSpec((c_blk, d_model), lambda j, i: (j, 0)),
        ],
        out_specs=pl.BlockSpec((1, 1), lambda j, i: (0, 0),
                               memory_space=pltpu.SMEM),
        out_shape=jax.ShapeDtypeStruct((1, 1), jnp.float32),
        scratch_shapes=[pltpu.VMEM((1, c_blk), jnp.float32)],
        interpret=interpret,
    )(cnl2, mr, zbb, cbb)


def _make_sc_gather(n_rows, d_model, n_workers, chunk):
    b_per_w = n_rows // n_workers
    n_chunks = b_per_w // chunk
    mesh = plsc.VectorSubcoreMesh(core_axis_name="c", subcore_axis_name="s")

    @functools.partial(
        pl.kernel, mesh=mesh,
        out_type=jax.ShapeDtypeStruct((n_rows, d_model), jnp.float32),
        scratch_types=[
            pltpu.VMEM((b_per_w,), jnp.int32),
            pltpu.VMEM((chunk, d_model), jnp.float32),
            pltpu.VMEM((chunk, d_model), jnp.float32),
            pltpu.SemaphoreType.DMA,
            pltpu.SemaphoreType.DMA,
        ],
    )
    def gather_k(cb_hbm, idx_hbm, out_hbm, idx_v, rows_a, rows_b, sem_a, sem_b):
        wid = lax.axis_index("s") * 2 + lax.axis_index("c")
        base = wid * b_per_w
        pltpu.sync_copy(idx_hbm.at[pl.ds(base, b_per_w)], idx_v)
        bufs = ((rows_a, sem_a), (rows_b, sem_b))
        cps = []
        for c in range(n_chunks):
            buf, sem = bufs[c % 2]
            cps.append(pltpu.async_copy(
                cb_hbm.at[idx_v.at[pl.ds(c * chunk, chunk)]], buf, sem))
            if c >= 1:
                cps[c - 1].wait()
                pbuf, _ = bufs[(c - 1) % 2]
                pltpu.sync_copy(
                    pbuf, out_hbm.at[pl.ds(base + (c - 1) * chunk, chunk)])
        cps[n_chunks - 1].wait()
        lbuf, _ = bufs[(n_chunks - 1) % 2]
        pltpu.sync_copy(
            lbuf, out_hbm.at[pl.ds(base + (n_chunks - 1) * chunk, chunk)])

    return gather_k


def kernel(z, codebook):
    b, k_seq, d_model = z.shape
    n = b * k_seq
    kcb = codebook.shape[0]
    z2 = z.reshape(n, d_model)

    cn = jnp.sum(codebook ** 2, axis=1)[None, :]
    cnl2 = cn * LOG2E
    cbb = codebook.astype(jnp.bfloat16)

    idx, mr, summ, zbb = _run_p1(z2, codebook, cn, _R1, _C1)

    gather_k = _make_sc_gather(n, d_model, 32, 128)
    z_q = gather_k(codebook, idx.reshape(n))

    ent = _run_p2(zbb, cbb, cnl2, mr, _R2, _C2)

    sum_min = summ[0, 0]
    entropy = ent[0, 0]
    max_ent = jnp.log(jnp.float32(kcb))
    total_loss = (1.25 * sum_min / jnp.float32(n * d_model)
                  + 0.1 * (max_ent - entropy) / max_ent)
    return (z_q.reshape(b, k_seq, d_model), total_loss,
            idx.reshape(b, k_seq))


# speedup vs baseline: 1.3418x; 1.1019x over previous
import functools

import jax
import jax.numpy as jnp
from jax import lax
from jax.experimental import pallas as pl
from jax.experimental.pallas import tpu as pltpu
from jax.experimental.pallas import tpu_sc as plsc

LOG2E = 1.4426950408889634

_R1, _C1 = 512, 8192
_R2, _C2 = 512, 8192


def _p1_body(cn_ref, z_ref, cb_ref, idx_ref, mr_ref, summ_ref, zbb_ref,
             zs_ref, zn_ref, m_ref, s_ref, fidx_ref,
             *, n_i, n_j, c_blk, k_total):
    i = pl.program_id(0)
    j = pl.program_id(1)

    @pl.when(j == 0)
    def _():
        zt = z_ref[...]
        zs_ref[...] = zt * (-2.0)
        zbb_ref[...] = (zt * (-2.0 * LOG2E)).astype(jnp.bfloat16)
        zn_ref[...] = jnp.sum(zt * zt, axis=1, keepdims=True)

    t = lax.dot_general(zs_ref[...], cb_ref[...],
                        dimension_numbers=(((1,), (1,)), ((), ())),
                        preferred_element_type=jnp.float32)
    d = (zn_ref[...] + t) + cn_ref[...]
    lmin = jnp.min(d, axis=1, keepdims=True)
    g = (lmin - d) * LOG2E
    jg = (lax.broadcasted_iota(jnp.int32, d.shape, 1).astype(jnp.float32)
          + jnp.float32(j * c_blk))
    larg = jnp.min(jnp.where(g == 0.0, jg, jnp.float32(2 * k_total)),
                   axis=1, keepdims=True)
    ls = jnp.sum(jnp.exp2(g), axis=1, keepdims=True)

    @pl.when(j == 0)
    def _():
        m_ref[...] = lmin
        fidx_ref[...] = larg
        s_ref[...] = ls

    @pl.when(j > 0)
    def _():
        mo = m_ref[...]
        so = s_ref[...]
        mn = jnp.minimum(mo, lmin)
        s_ref[...] = (so * jnp.exp2((mn - mo) * LOG2E)
                      + ls * jnp.exp2((mn - lmin) * LOG2E))
        m_ref[...] = mn
        fidx_ref[...] = jnp.where(lmin < mo, larg, fidx_ref[...])

    @pl.when(j == n_j - 1)
    def _():
        part = jnp.sum(m_ref[...])
        idx_ref[...] = fidx_ref[...].astype(jnp.int32)
        mr_ref[...] = ((m_ref[...] - zn_ref[...])
                       - jnp.log(s_ref[...])) * LOG2E

        @pl.when(i == 0)
        def _():
            summ_ref[0, 0] = part

        @pl.when(i > 0)
        def _():
            summ_ref[0, 0] = summ_ref[0, 0] + part


def _p2_body(cnl2_ref, mr_ref, z_ref, cb_ref, ent_ref, acc_ref,
             *, n_i, n_j, n_rows):
    j = pl.program_id(0)
    i = pl.program_id(1)
    t = lax.dot_general(z_ref[...], cb_ref[...],
                        dimension_numbers=(((1,), (1,)), ((), ())),
                        preferred_element_type=jnp.float32)
    w = jnp.exp2(mr_ref[...] - (t + cnl2_ref[...]))
    cs = jnp.sum(w, axis=0, keepdims=True)

    @pl.when(i == 0)
    def _():
        acc_ref[...] = cs

    @pl.when(i > 0)
    def _():
        acc_ref[...] = acc_ref[...] + cs

    @pl.when(i == n_i - 1)
    def _():
        p = acc_ref[...] * (1.0 / n_rows)
        part = -jnp.sum(p * jnp.log(p + 1e-10))

        @pl.when(j == 0)
        def _():
            ent_ref[0, 0] = part

        @pl.when(j > 0)
        def _():
            ent_ref[0, 0] = ent_ref[0, 0] + part


def _run_p1(z2, codebook, cn, r_blk, c_blk, interpret=False):
    n, d_model = z2.shape
    kcb = codebook.shape[0]
    n_i, n_j = n // r_blk, kcb // c_blk
    return pl.pallas_call(
        functools.partial(_p1_body, n_i=n_i, n_j=n_j, c_blk=c_blk,
                          k_total=kcb),
        grid=(n_i, n_j),
        in_specs=[
            pl.BlockSpec((1, c_blk), lambda i, j: (0, j)),
            pl.BlockSpec((r_blk, d_model), lambda i, j: (i, 0)),
            pl.BlockSpec((c_blk, d_model), lambda i, j: (j, 0)),
        ],
        out_specs=[
            pl.BlockSpec((r_blk, 1), lambda i, j: (i, 0)),
            pl.BlockSpec((r_blk, 1), lambda i, j: (i, 0)),
            pl.BlockSpec((1, 1), lambda i, j: (0, 0),
                         memory_space=pltpu.SMEM),
            pl.BlockSpec((r_blk, d_model), lambda i, j: (i, 0)),
        ],
        out_shape=[
            jax.ShapeDtypeStruct((n, 1), jnp.int32),
            jax.ShapeDtypeStruct((n, 1), jnp.float32),
            jax.ShapeDtypeStruct((1, 1), jnp.float32),
            jax.ShapeDtypeStruct((n, d_model), jnp.bfloat16),
        ],
        scratch_shapes=[
            pltpu.VMEM((r_blk, d_model), jnp.float32),
            pltpu.VMEM((r_blk, 1), jnp.float32),
            pltpu.VMEM((r_blk, 1), jnp.float32),
            pltpu.VMEM((r_blk, 1), jnp.float32),
            pltpu.VMEM((r_blk, 1), jnp.float32),
        ],
        interpret=interpret,
    )(cn, z2, codebook)


def _run_p2(zbb, cbb, cnl2, mr, r_blk, c_blk, interpret=False):
    n, d_model = zbb.shape
    kcb = cbb.shape[0]
    n_i, n_j = n // r_blk, kcb // c_blk
    return pl.pallas_call(
        functools.partial(_p2_body, n_i=n_i, n_j=n_j, n_rows=n),
        grid=(n_j, n_i),
        in_specs=[
            pl.BlockSpec((1, c_blk), lambda j, i: (0, j)),
            pl.BlockSpec((r_blk, 1), lambda j, i: (i, 0)),
            pl.BlockSpec((r_blk, d_model), lambda j, i: (i, 0)),
            pl.BlockSpec((c_blk, d_model), lambda j, i: (j, 0)),
        ],
        out_specs=pl.BlockSpec((1, 1), lambda j, i: (0, 0),
                               memory_space=pltpu.SMEM),
        out_shape=jax.ShapeDtypeStruct((1, 1), jnp.float32),
        scratch_shapes=[pltpu.VMEM((1, c_blk), jnp.float32)],
        interpret=interpret,
    )(cnl2, mr, zbb, cbb)


def _make_sc_gather(n_rows, d_model, n_workers, chunk):
    b_per_w = n_rows // n_workers
    n_chunks = b_per_w // chunk
    mesh = plsc.VectorSubcoreMesh(core_axis_name="c", subcore_axis_name="s")

    @functools.partial(
        pl.kernel, mesh=mesh,
        out_type=jax.ShapeDtypeStruct((n_rows, d_model), jnp.float32),
        scratch_types=[
            pltpu.VMEM((b_per_w,), jnp.int32),
            pltpu.VMEM((chunk, d_model), jnp.float32),
            pltpu.VMEM((chunk, d_model), jnp.float32),
            pltpu.SemaphoreType.DMA,
            pltpu.SemaphoreType.DMA,
        ],
    )
    def gather_k(cb_hbm, idx_hbm, out_hbm, idx_v, rows_a, rows_b, sem_a, sem_b):
        wid = lax.axis_index("s") * 2 + lax.axis_index("c")
        base = wid * b_per_w
        pltpu.sync_copy(idx_hbm.at[pl.ds(base, b_per_w)], idx_v)
        bufs = ((rows_a, sem_a), (rows_b, sem_b))
        cps = []
        for c in range(n_chunks):
            buf, sem = bufs[c % 2]
            cps.append(pltpu.async_copy(
                cb_hbm.at[idx_v.at[pl.ds(c * chunk, chunk)]], buf, sem))
            if c >= 1:
                cps[c - 1].wait()
                pbuf, _ = bufs[(c - 1) % 2]
                pltpu.sync_copy(
                    pbuf, out_hbm.at[pl.ds(base + (c - 1) * chunk, chunk)])
        cps[n_chunks - 1].wait()
        lbuf, _ = bufs[(n_chunks - 1) % 2]
        pltpu.sync_copy(
            lbuf, out_hbm.at[pl.ds(base + (n_chunks - 1) * chunk, chunk)])

    return gather_k


def kernel(z, codebook):
    b, k_seq, d_model = z.shape
    n = b * k_seq
    kcb = codebook.shape[0]
    z2 = z.reshape(n, d_model)

    cn = jnp.sum(codebook ** 2, axis=1)[None, :]
    cnl2 = cn * LOG2E
    cbb = codebook.astype(jnp.bfloat16)

    idx, mr, summ, zbb = _run_p1(z2, codebook, cn, _R1, _C1)

    gather_k = _make_sc_gather(n, d_model, 32, 128)
    z_q = gather_k(codebook, idx.reshape(n))

    ent = _run_p2(zbb, cbb, cnl2, mr, _R2, _C2)

    sum_min = summ[0, 0]
    entropy = ent[0, 0]
    max_ent = jnp.log(jnp.float32(kcb))
    total_loss = (1.25 * sum_min / jnp.float32(n * d_model)
                  + 0.1 * (max_ent - entropy) / max_ent)
    return (z_q.reshape(b, k_seq, d_model), total_loss,
            idx.reshape(b, k_seq))
